# baseline (speedup 1.0000x reference)
import functools

import jax
import jax.numpy as jnp
from jax import lax
from jax.experimental import pallas as pl
from jax.experimental.pallas import tpu as pltpu
from jax.experimental.pallas import tpu_sc as plsc

N = 10000
NH = N // 2
E = 320000
NC = 2
NS = 16
NW = NC * NS
CH = 80
NB = 8
B = NB * CH
NBLK = E // B

_mesh = plsc.VectorSubcoreMesh(
    core_axis_name="c", subcore_axis_name="s", num_cores=NC, num_subcores=NS)
_params = pltpu.CompilerParams(needs_layout_passes=False)


def _f32(shape):
    return jax.ShapeDtypeStruct(shape, jnp.float32)


def _per_subcore_slice(s, total, copyfn):
    big = (total // NS + 7) // 8 * 8
    last = total - (NS - 1) * big

    @pl.when(s < NS - 1)
    def _():
        copyfn(s * big, big)

    @pl.when(s == NS - 1)
    def _():
        copyfn((NS - 1) * big, last)


def _tc1_body(x_ref, w_ref, a_ref, h_ref, att_ref, bnd_ref):
    i = pl.program_id(0)
    h = jnp.dot(x_ref[...], w_ref[...], preferred_element_type=jnp.float32)
    att = jnp.dot(h, a_ref[...], preferred_element_type=jnp.float32)
    h_ref[...] = h
    att_ref[...] = att
    bm = jnp.broadcast_to(jnp.max(att, axis=0, keepdims=True), (8, 128))

    @pl.when(i == 0)
    def _():
        bnd_ref[...] = bm

    @pl.when(i != 0)
    def _():
        bnd_ref[...] = jnp.maximum(bnd_ref[...], bm)


def _tc1(x, W1, A1):
    return pl.pallas_call(
        _tc1_body,
        grid=(25,),
        in_specs=[
            pl.BlockSpec((400, 128), lambda i: (i, 0)),
            pl.BlockSpec((128, 1024), lambda i: (0, 0)),
            pl.BlockSpec((1024, 128), lambda i: (0, 0)),
        ],
        out_specs=[
            pl.BlockSpec((400, 1024), lambda i: (i, 0)),
            pl.BlockSpec((400, 128), lambda i: (i, 0)),
            pl.BlockSpec((8, 128), lambda i: (0, 0)),
        ],
        out_shape=[_f32((N, 1024)), _f32((N, 128)), _f32((8, 128))],
    )(x, W1, A1)


def _sca_body(src_h, dst_h, att_h, bnd_h, zden_h,
              wt_h, den_h,
              src_v, dst_v, asrc, adst, wtv, bnd_v, den_t, sem):
    c = lax.axis_index("c")
    s = lax.axis_index("s")
    wid = c * NS + s
    pltpu.sync_copy(zden_h, den_t)
    pltpu.sync_copy(bnd_h, bnd_v)
    bvecs = [bnd_v[h, pl.ds(0, 16)] for h in range(8)]

    def blk(k, carry):
        bg = wid + NW * k
        row = bg * NB
        pltpu.sync_copy(src_h.at[pl.ds(row, NB)], src_v)
        pltpu.sync_copy(dst_h.at[pl.ds(row, NB)], dst_v)
        for i in range(NB):
            d1 = pltpu.async_copy(att_h.at[src_v.at[i]], asrc, sem)
            d2 = pltpu.async_copy(att_h.at[dst_v.at[i]], adst, sem)
            d1.wait()
            d2.wait()
            for j in range(5):
                eidx = lax.iota(jnp.int32, 16) + j * 16
                dstv = dst_v[i, pl.ds(j * 16, 16)]
                zrow = jnp.full((16,), 0, jnp.int32)
                for h in range(8):
                    hv = jnp.full((16,), h, jnp.int32)
                    sv = plsc.load_gather(asrc, [eidx, hv])
                    dv = plsc.load_gather(adst, [eidx, jnp.full((16,), 8 + h,
                                                                jnp.int32)])
                    t = sv + dv
                    t = jnp.maximum(t, 0.2 * t)
                    w = jnp.exp(t - bvecs[h])
                    wtv[h, pl.ds(i * CH + j * 16, 16)] = w
                    plsc.addupdate_scatter(den_t, [zrow, dstv * 8 + h], w)
        pltpu.sync_copy(wtv, wt_h.at[:, pl.ds(bg * B, B)])
        return carry

    cnt = jnp.where(wid < NBLK - 15 * NW, 16, 15)
    lax.fori_loop(0, cnt, blk, 0)
    pltpu.sync_copy(den_t, den_h.at[wid])


_sca = functools.partial(
    pl.kernel,
    _sca_body,
    out_type=(_f32((8, E)), _f32((NW, 1, 8 * N))),
    mesh=_mesh,
    compiler_params=_params,
    scratch_types=[
        pltpu.VMEM((NB, CH), jnp.int32),
        pltpu.VMEM((NB, CH), jnp.int32),
        pltpu.VMEM((CH, 128), jnp.float32),
        pltpu.VMEM((CH, 128), jnp.float32),
        pltpu.VMEM((8, B), jnp.float32),
        pltpu.VMEM((8, 16), jnp.float32),
        pltpu.VMEM((1, 8 * N), jnp.float32),
        pltpu.SemaphoreType.DMA,
    ],
)()


def _scb_body(src_h, dst_h, wt_h, h1_h, z128_h,
              out_h,
              src_v, dst_v, idx_v, w_v, hbuf, hbuf2, acc_sp, gsem, ssem):
    c = lax.axis_index("c")
    s = lax.axis_index("s")
    for hl in range(4):
        hg = c * 4 + hl
        _per_subcore_slice(s, N, lambda off, n: pltpu.sync_copy(
            z128_h.at[pl.ds(off, n)], acc_sp.at[pl.ds(off, n)]))
        plsc.subcore_barrier()

        def blk(k, carry):
            bg = s + NS * k
            row = bg * NB
            pltpu.sync_copy(src_h.at[pl.ds(row, NB)], src_v)
            pltpu.sync_copy(dst_h.at[pl.ds(row, NB)], dst_v)
            pltpu.sync_copy(wt_h.at[hg, pl.ds(row, NB)], w_v)
            for j in range(B // 16):
                r, q = j // 5, (j % 5) * 16
                sv = src_v[r, pl.ds(q, 16)]
                idx_v[r, pl.ds(q, 16)] = sv * 8 + hg

            def scale(buf, i):
                iv = jnp.full((16,), i, jnp.int32)

                def rowfn(r, cr):
                    for u in range(4):
                        wb = plsc.load_gather(
                            w_v, [iv, jnp.broadcast_to(4 * r + u, (16,))])
                        for kk in range(8):
                            buf[4 * r + u, pl.ds(kk * 16, 16)] = (
                                buf[4 * r + u, pl.ds(kk * 16, 16)] * wb)
                    return cr

                lax.fori_loop(0, CH // 4, rowfn, 0)

            bufs = (hbuf, hbuf2)
            gd = [None] * NB
            sd = [None] * NB
            gd[0] = pltpu.async_copy(h1_h.at[idx_v.at[0]], bufs[0], gsem[0])
            for i in range(1, NB + 1):
                if i >= 2:
                    sd[i - 2].wait()
                if i < NB:
                    gd[i] = pltpu.async_copy(
                        h1_h.at[idx_v.at[i]], bufs[i % 2], gsem[i % 2])
                gd[i - 1].wait()
                scale(bufs[(i - 1) % 2], i - 1)
                sd[i - 1] = pltpu.async_copy(
                    bufs[(i - 1) % 2], acc_sp.at[dst_v.at[i - 1]],
                    ssem[(i - 1) % 2], add=True)
            sd[NB - 1].wait()
            return carry

        cnt = jnp.where(s < NBLK - 31 * NS, 32, 31)
        lax.fori_loop(0, cnt, blk, 0)
        plsc.subcore_barrier()
        _per_subcore_slice(s, N, lambda off, n: pltpu.sync_copy(
            acc_sp.at[pl.ds(off, n)], out_h.at[hg, pl.ds(off, n)]))
        plsc.subcore_barrier()


_scb = functools.partial(
    pl.kernel,
    _scb_body,
    out_type=_f32((8, N, 128)),
    mesh=_mesh,
    compiler_params=_params,
    scratch_types=[
        pltpu.VMEM((NB, CH), jnp.int32),
        pltpu.VMEM((NB, CH), jnp.int32),
        pltpu.VMEM((NB, CH), jnp.int32),
        pltpu.VMEM((NB, CH), jnp.float32),
        pltpu.VMEM((CH, 128), jnp.float32),
        pltpu.VMEM((CH, 128), jnp.float32),
        pltpu.VMEM_SHARED((N, 128), jnp.float32),
        (pltpu.SemaphoreType.DMA, pltpu.SemaphoreType.DMA),
        (pltpu.SemaphoreType.DMA, pltpu.SemaphoreType.DMA),
    ],
)()


def _tc2_body(o1_ref, den_ref, att_ref, h1_ref, w2_ref, a2_ref, b1_ref,
              bsh_ref, h2_ref, att2_ref, bnd_ref):
    i = pl.program_id(0)
    att = att_ref[...]
    pieces = []
    for h in range(8):
        t = att[:, h:h + 1] + att[:, 8 + h:9 + h]
        t = jnp.maximum(t, 0.2 * t)
        sw = jnp.exp(t - bsh_ref[0, h])
        den = jnp.sum(den_ref[:, :, h:h + 1], axis=0) + sw + 1e-16
        num = o1_ref[h] + sw * h1_ref[:, h * 128:(h + 1) * 128]
        o = num / den + b1_ref[:, h * 128:(h + 1) * 128]
        pieces.append(jnp.where(o > 0, o, jnp.exp(jnp.minimum(o, 0.0)) - 1.0))
    x2 = jnp.concatenate(pieces, axis=1)
    h2 = jnp.dot(x2, w2_ref[...], preferred_element_type=jnp.float32)
    att2 = jnp.dot(h2, a2_ref[...], preferred_element_type=jnp.float32)
    h2_ref[...] = h2
    att2_ref[...] = att2
    bm = jnp.broadcast_to(jnp.max(att2, axis=0, keepdims=True), (8, 128))

    @pl.when(i == 0)
    def _():
        bnd_ref[...] = bm

    @pl.when(i != 0)
    def _():
        bnd_ref[...] = jnp.maximum(bnd_ref[...], bm)


def _tc2(out1, den1p, att1, h1, W2, A2, b1, bsh1):
    return pl.pallas_call(
        _tc2_body,
        grid=(25,),
        in_specs=[
            pl.BlockSpec((8, 400, 128), lambda i: (0, i, 0)),
            pl.BlockSpec((NW, 400, 8), lambda i: (0, i, 0)),
            pl.BlockSpec((400, 128), lambda i: (i, 0)),
            pl.BlockSpec((400, 1024), lambda i: (i, 0)),
            pl.BlockSpec((1024, 128), lambda i: (0, 0)),
            pl.BlockSpec((128, 128), lambda i: (0, 0)),
            pl.BlockSpec((1, 1024), lambda i: (0, 0)),
            pl.BlockSpec((1, 16), lambda i: (0, 0)),
        ],
        out_specs=[
            pl.BlockSpec((400, 128), lambda i: (i, 0)),
            pl.BlockSpec((400, 128), lambda i: (i, 0)),
            pl.BlockSpec((8, 128), lambda i: (0, 0)),
        ],
        out_shape=[_f32((N, 128)), _f32((N, 128)), _f32((8, 128))],
    )(out1, den1p, att1, h1, W2, A2, b1, bsh1)


def _scc_body(src_h, dst_h, att_h, h2_h, bnd_h, z128_h, zd2_h,
              acc_h, den_h,
              src_v, dst_v, idx_v, w_v, asrc, adst, hbuf, bnd_v, den_t,
              acc_sp, sem):
    c = lax.axis_index("c")
    s = lax.axis_index("s")
    wid = c * NS + s
    _per_subcore_slice(s, NH, lambda off, n: pltpu.sync_copy(
        z128_h.at[pl.ds(off, n)], acc_sp.at[pl.ds(off, n)]))
    pltpu.sync_copy(zd2_h, den_t)
    pltpu.sync_copy(bnd_h, bnd_v)
    plsc.subcore_barrier()
    bvec = bnd_v[...]
    zv = jnp.full((16,), 0, jnp.int32)
    ov = jnp.full((16,), 1, jnp.int32)
    base = c * NH

    def blk(k, carry):
        bg = s + NS * k
        row = bg * NB
        pltpu.sync_copy(src_h.at[pl.ds(row, NB)], src_v)
        pltpu.sync_copy(dst_h.at[pl.ds(row, NB)], dst_v)
        for i in range(NB):
            d1 = pltpu.async_copy(att_h.at[src_v.at[i]], asrc, sem)
            d2 = pltpu.async_copy(att_h.at[dst_v.at[i]], adst, sem)
            d3 = pltpu.async_copy(h2_h.at[src_v.at[i]], hbuf, sem)
            d1.wait()
            d2.wait()
            d3.wait()
            for j in range(5):
                eidx = lax.iota(jnp.int32, 16) + j * 16
                dstv = dst_v[i, pl.ds(j * 16, 16)]
                lidx = dstv - base
                inh = (lidx >= 0) & (lidx < NH)
                sv = plsc.load_gather(asrc, [eidx, zv])
                dv = plsc.load_gather(adst, [eidx, ov])
                t = sv + dv
                t = jnp.maximum(t, 0.2 * t)
                w = jnp.exp(t - bvec) * inh.astype(jnp.float32)
                w_v[0, pl.ds(j * 16, 16)] = w
                idx_v[i, pl.ds(j * 16, 16)] = jnp.where(inh, lidx, 0)
                plsc.addupdate_scatter(den_t, [zv, dstv], w)

            def rowfn(r, cr):
                wb = plsc.load_gather(
                    w_v, [jnp.broadcast_to(r // CH, (16,)),
                          jnp.broadcast_to(r % CH, (16,))])
                for kk in range(8):
                    hbuf[r, pl.ds(kk * 16, 16)] = (
                        hbuf[r, pl.ds(kk * 16, 16)] * wb)
                return cr

            lax.fori_loop(0, CH, rowfn, 0)
            pltpu.sync_copy(hbuf, acc_sp.at[idx_v.at[i]], add=True)
        return carry

    cnt = jnp.where(s < NBLK - 31 * NS, 32, 31)
    lax.fori_loop(0, cnt, blk, 0)
    pltpu.sync_copy(den_t, den_h.at[wid])
    plsc.subcore_barrier()
    _per_subcore_slice(s, NH, lambda off, n: pltpu.sync_copy(
        acc_sp.at[pl.ds(off, n)], acc_h.at[pl.ds(base + off, n)]))


_scc = functools.partial(
    pl.kernel,
    _scc_body,
    out_type=(_f32((N, 128)), _f32((NW, 1, N))),
    mesh=_mesh,
    compiler_params=_params,
    scratch_types=[
        pltpu.VMEM((NB, CH), jnp.int32),
        pltpu.VMEM((NB, CH), jnp.int32),
        pltpu.VMEM((NB, CH), jnp.int32),
        pltpu.VMEM((1, CH), jnp.float32),
        pltpu.VMEM((CH, 128), jnp.float32),
        pltpu.VMEM((CH, 128), jnp.float32),
        pltpu.VMEM((CH, 128), jnp.float32),
        pltpu.VMEM((16,), jnp.float32),
        pltpu.VMEM((1, N), jnp.float32),
        pltpu.VMEM_SHARED((NH, 128), jnp.float32),
        pltpu.SemaphoreType.DMA,
    ],
)()


def _tc3_body(acc_ref, den_ref, att2_ref, h2_ref, b2_ref, bnd_ref, out_ref):
    t = att2_ref[:, 0:1] + att2_ref[:, 1:2]
    t = jnp.maximum(t, 0.2 * t)
    sw = jnp.exp(t - bnd_ref[0, 0])
    den = jnp.sum(den_ref[...], axis=1, keepdims=True) + sw + 1e-16
    out_ref[...] = ((acc_ref[...] + sw * h2_ref[...]) / den + b2_ref[...])


def _tc3(acc2, den2p, att2, h2, b2, bnd2):
    return pl.pallas_call(
        _tc3_body,
        grid=(25,),
        in_specs=[
            pl.BlockSpec((400, 128), lambda i: (i, 0)),
            pl.BlockSpec((400, NW), lambda i: (i, 0)),
            pl.BlockSpec((400, 128), lambda i: (i, 0)),
            pl.BlockSpec((400, 128), lambda i: (i, 0)),
            pl.BlockSpec((1, 128), lambda i: (0, 0)),
            pl.BlockSpec((1, 1), lambda i: (0, 0)),
        ],
        out_specs=pl.BlockSpec((400, 128), lambda i: (i, 0)),
        out_shape=_f32((N, 128)),
    )(acc2, den2p, att2, h2, b2, bnd2)


def kernel(x, edge_index, W1, att_src1, att_dst1, b1,
           W2, att_src2, att_dst2, b2):
    src = edge_index[0].astype(jnp.int32).reshape(E // CH, CH)
    dst = edge_index[1].astype(jnp.int32).reshape(E // CH, CH)

    rows1 = jnp.arange(1024)
    A1 = jnp.zeros((1024, 128), jnp.float32)
    A1 = A1.at[rows1, rows1 // 128].set(att_src1.reshape(-1))
    A1 = A1.at[rows1, 8 + rows1 // 128].set(att_dst1.reshape(-1))
    rows2 = jnp.arange(128)
    A2 = jnp.zeros((128, 128), jnp.float32)
    A2 = A2.at[rows2, 0].set(att_src2.reshape(-1))
    A2 = A2.at[rows2, 1].set(att_dst2.reshape(-1))

    zden = jnp.zeros((1, 8 * N), jnp.float32)
    zd2 = jnp.zeros((1, N), jnp.float32)
    z128 = jnp.zeros((N, 128), jnp.float32)

    h1, att1, bnd1 = _tc1(x, W1, A1)
    bv1 = jnp.max(bnd1, axis=0)
    bsh8 = bv1[:8] + bv1[8:16]
    bsh1 = jnp.concatenate([bsh8, jnp.zeros((8,), jnp.float32)])
    wt, den1p = _sca(src, dst, att1,
                     jnp.broadcast_to(bsh8[:, None], (8, 16)), zden)
    den1p = den1p.reshape(NW, N, 8)
    out1 = _scb(src, dst, wt.reshape(8, E // CH, CH),
                h1.reshape(N * 8, 128), z128)
    h2, att2, bnd2 = _tc2(out1, den1p, att1, h1, W2, A2,
                          b1.reshape(1, 1024), bsh1.reshape(1, 16))
    bv2 = jnp.max(bnd2, axis=0)
    bound2 = bv2[0] + bv2[1]
    acc2, den2p = _scc(src, dst, att2, h2, jnp.full((16,), bound2),
                       z128, zd2)
    den2t = den2p.reshape(NW, N).T
    return _tc3(acc2, den2t, att2, h2, b2.reshape(1, 128),
                bound2.reshape(1, 1))


# speedup vs baseline: 1.0944x; 1.0944x over previous
import functools

import jax
import jax.numpy as jnp
from jax import lax
from jax.experimental import pallas as pl
from jax.experimental.pallas import tpu as pltpu
from jax.experimental.pallas import tpu_sc as plsc

N = 10000
NH = N // 2
E = 320000
NC = 2
NS = 16
NW = NC * NS
CH = 80
NB = 8
B = NB * CH
NBLK = E // B

_mesh = plsc.VectorSubcoreMesh(
    core_axis_name="c", subcore_axis_name="s", num_cores=NC, num_subcores=NS)
_params = pltpu.CompilerParams(needs_layout_passes=False)


def _f32(shape):
    return jax.ShapeDtypeStruct(shape, jnp.float32)


def _per_subcore_slice(s, total, copyfn):
    big = (total // NS + 7) // 8 * 8
    last = total - (NS - 1) * big

    @pl.when(s < NS - 1)
    def _():
        copyfn(s * big, big)

    @pl.when(s == NS - 1)
    def _():
        copyfn((NS - 1) * big, last)


def _tc1_body(x_ref, w_ref, a_ref, h_ref, att_ref, bnd_ref):
    i = pl.program_id(0)
    h = jnp.dot(x_ref[...], w_ref[...], preferred_element_type=jnp.float32)
    att = jnp.dot(h, a_ref[...], preferred_element_type=jnp.float32)
    h_ref[...] = h
    att_ref[...] = att
    bm = jnp.broadcast_to(jnp.max(att, axis=0, keepdims=True), (8, 128))

    @pl.when(i == 0)
    def _():
        bnd_ref[...] = bm

    @pl.when(i != 0)
    def _():
        bnd_ref[...] = jnp.maximum(bnd_ref[...], bm)


def _tc1(x, W1, A1):
    return pl.pallas_call(
        _tc1_body,
        grid=(25,),
        in_specs=[
            pl.BlockSpec((400, 128), lambda i: (i, 0)),
            pl.BlockSpec((128, 1024), lambda i: (0, 0)),
            pl.BlockSpec((1024, 128), lambda i: (0, 0)),
        ],
        out_specs=[
            pl.BlockSpec((400, 1024), lambda i: (i, 0)),
            pl.BlockSpec((400, 128), lambda i: (i, 0)),
            pl.BlockSpec((8, 128), lambda i: (0, 0)),
        ],
        out_shape=[_f32((N, 1024)), _f32((N, 128)), _f32((8, 128))],
    )(x, W1, A1)


def _sca_body(src_h, dst_h, att_h, bnd_h, zden_h,
              wt_h, den_h,
              src_v, dst_v, asrc, adst, wtv, bnd_v, den_t, sem):
    c = lax.axis_index("c")
    s = lax.axis_index("s")
    wid = c * NS + s
    pltpu.sync_copy(zden_h, den_t)
    pltpu.sync_copy(bnd_h, bnd_v)
    bvecs = [bnd_v[h, pl.ds(0, 16)] for h in range(8)]

    def blk(k, carry):
        bg = wid + NW * k
        row = bg * NB
        pltpu.sync_copy(src_h.at[pl.ds(row, NB)], src_v)
        pltpu.sync_copy(dst_h.at[pl.ds(row, NB)], dst_v)
        for i in range(NB):
            d1 = pltpu.async_copy(att_h.at[src_v.at[i]], asrc, sem)
            d2 = pltpu.async_copy(att_h.at[dst_v.at[i]], adst, sem)
            d1.wait()
            d2.wait()
            for j in range(5):
                eidx = lax.iota(jnp.int32, 16) + j * 16
                dstv = dst_v[i, pl.ds(j * 16, 16)]
                zrow = jnp.full((16,), 0, jnp.int32)
                for h in range(8):
                    hv = jnp.full((16,), h, jnp.int32)
                    sv = plsc.load_gather(asrc, [eidx, hv])
                    dv = plsc.load_gather(adst, [eidx, jnp.full((16,), 8 + h,
                                                                jnp.int32)])
                    t = sv + dv
                    t = jnp.maximum(t, 0.2 * t)
                    w = jnp.exp(t - bvecs[h])
                    wtv[h, pl.ds(i * CH + j * 16, 16)] = w
                    plsc.addupdate_scatter(den_t, [zrow, dstv * 8 + h], w)
        pltpu.sync_copy(wtv, wt_h.at[:, pl.ds(bg * B, B)])
        return carry

    cnt = jnp.where(wid < NBLK - 15 * NW, 16, 15)
    lax.fori_loop(0, cnt, blk, 0)
    pltpu.sync_copy(den_t, den_h.at[wid])


_sca = functools.partial(
    pl.kernel,
    _sca_body,
    out_type=(_f32((8, E)), _f32((NW, 1, 8 * N))),
    mesh=_mesh,
    compiler_params=_params,
    scratch_types=[
        pltpu.VMEM((NB, CH), jnp.int32),
        pltpu.VMEM((NB, CH), jnp.int32),
        pltpu.VMEM((CH, 128), jnp.float32),
        pltpu.VMEM((CH, 128), jnp.float32),
        pltpu.VMEM((8, B), jnp.float32),
        pltpu.VMEM((8, 16), jnp.float32),
        pltpu.VMEM((1, 8 * N), jnp.float32),
        pltpu.SemaphoreType.DMA,
    ],
)()


def _scb_body(src_h, dst_h, wt_h, h1_h, z128_h,
              out_h,
              src_v, dst_v, idx_v, w_v, hbuf, hbuf2, acc_sp, gsem, ssem):
    c = lax.axis_index("c")
    s = lax.axis_index("s")
    for hl in range(4):
        hg = c * 4 + hl
        _per_subcore_slice(s, N, lambda off, n: pltpu.sync_copy(
            z128_h.at[pl.ds(off, n)], acc_sp.at[pl.ds(off, n)]))
        plsc.subcore_barrier()

        def blk(k, carry):
            bg = s + NS * k
            row = bg * NB
            pltpu.sync_copy(src_h.at[pl.ds(row, NB)], src_v)
            pltpu.sync_copy(dst_h.at[pl.ds(row, NB)], dst_v)
            pltpu.sync_copy(wt_h.at[hg, pl.ds(row, NB)], w_v)
            for j in range(B // 16):
                r, q = j // 5, (j % 5) * 16
                sv = src_v[r, pl.ds(q, 16)]
                idx_v[r, pl.ds(q, 16)] = sv * 8 + hg

            def scale(buf, i):
                iv = jnp.full((16,), i, jnp.int32)

                def rowfn(r, cr):
                    for u in range(4):
                        wb = plsc.load_gather(
                            w_v, [iv, jnp.broadcast_to(4 * r + u, (16,))])
                        for kk in range(8):
                            buf[4 * r + u, pl.ds(kk * 16, 16)] = (
                                buf[4 * r + u, pl.ds(kk * 16, 16)] * wb)
                    return cr

                lax.fori_loop(0, CH // 4, rowfn, 0)

            bufs = (hbuf, hbuf2)
            gd = [None] * NB
            sd = [None] * NB
            gd[0] = pltpu.async_copy(h1_h.at[idx_v.at[0]], bufs[0], gsem[0])
            for i in range(1, NB + 1):
                if i >= 2:
                    sd[i - 2].wait()
                if i < NB:
                    gd[i] = pltpu.async_copy(
                        h1_h.at[idx_v.at[i]], bufs[i % 2], gsem[i % 2])
                gd[i - 1].wait()
                scale(bufs[(i - 1) % 2], i - 1)
                sd[i - 1] = pltpu.async_copy(
                    bufs[(i - 1) % 2], acc_sp.at[dst_v.at[i - 1]],
                    ssem[(i - 1) % 2], add=True)
            sd[NB - 1].wait()
            return carry

        cnt = jnp.where(s < NBLK - 31 * NS, 32, 31)
        lax.fori_loop(0, cnt, blk, 0)
        plsc.subcore_barrier()
        _per_subcore_slice(s, N, lambda off, n: pltpu.sync_copy(
            acc_sp.at[pl.ds(off, n)], out_h.at[hg, pl.ds(off, n)]))
        plsc.subcore_barrier()


_scb = functools.partial(
    pl.kernel,
    _scb_body,
    out_type=_f32((8, N, 128)),
    mesh=_mesh,
    compiler_params=_params,
    scratch_types=[
        pltpu.VMEM((NB, CH), jnp.int32),
        pltpu.VMEM((NB, CH), jnp.int32),
        pltpu.VMEM((NB, CH), jnp.int32),
        pltpu.VMEM((NB, CH), jnp.float32),
        pltpu.VMEM((CH, 128), jnp.float32),
        pltpu.VMEM((CH, 128), jnp.float32),
        pltpu.VMEM_SHARED((N, 128), jnp.float32),
        (pltpu.SemaphoreType.DMA, pltpu.SemaphoreType.DMA),
        (pltpu.SemaphoreType.DMA, pltpu.SemaphoreType.DMA),
    ],
)()


def _tc2_body(o1_ref, den_ref, att_ref, h1_ref, w2_ref, a2_ref, b1_ref,
              bsh_ref, h2_ref, att2_ref, bnd_ref):
    i = pl.program_id(0)
    att = att_ref[...]
    pieces = []
    for h in range(8):
        t = att[:, h:h + 1] + att[:, 8 + h:9 + h]
        t = jnp.maximum(t, 0.2 * t)
        sw = jnp.exp(t - bsh_ref[0, h])
        den = jnp.sum(den_ref[:, :, h:h + 1], axis=0) + sw + 1e-16
        num = o1_ref[h] + sw * h1_ref[:, h * 128:(h + 1) * 128]
        o = num / den + b1_ref[:, h * 128:(h + 1) * 128]
        pieces.append(jnp.where(o > 0, o, jnp.exp(jnp.minimum(o, 0.0)) - 1.0))
    x2 = jnp.concatenate(pieces, axis=1)
    h2 = jnp.dot(x2, w2_ref[...], preferred_element_type=jnp.float32)
    att2 = jnp.dot(h2, a2_ref[...], preferred_element_type=jnp.float32)
    h2_ref[...] = h2
    att2_ref[...] = att2
    bm = jnp.broadcast_to(jnp.max(att2, axis=0, keepdims=True), (8, 128))

    @pl.when(i == 0)
    def _():
        bnd_ref[...] = bm

    @pl.when(i != 0)
    def _():
        bnd_ref[...] = jnp.maximum(bnd_ref[...], bm)


def _tc2(out1, den1p, att1, h1, W2, A2, b1, bsh1):
    return pl.pallas_call(
        _tc2_body,
        grid=(25,),
        in_specs=[
            pl.BlockSpec((8, 400, 128), lambda i: (0, i, 0)),
            pl.BlockSpec((NW, 400, 8), lambda i: (0, i, 0)),
            pl.BlockSpec((400, 128), lambda i: (i, 0)),
            pl.BlockSpec((400, 1024), lambda i: (i, 0)),
            pl.BlockSpec((1024, 128), lambda i: (0, 0)),
            pl.BlockSpec((128, 128), lambda i: (0, 0)),
            pl.BlockSpec((1, 1024), lambda i: (0, 0)),
            pl.BlockSpec((1, 16), lambda i: (0, 0)),
        ],
        out_specs=[
            pl.BlockSpec((400, 128), lambda i: (i, 0)),
            pl.BlockSpec((400, 128), lambda i: (i, 0)),
            pl.BlockSpec((8, 128), lambda i: (0, 0)),
        ],
        out_shape=[_f32((N, 128)), _f32((N, 128)), _f32((8, 128))],
    )(out1, den1p, att1, h1, W2, A2, b1, bsh1)


def _scc_body(src_h, dst_h, att_h, h2_h, bnd_h, z128_h, zd2_h,
              acc_h, den_h,
              src_v, dst_v, idx_v, w_v, asrc, adst, hbuf, bnd_v, den_t,
              acc_sp, gsem, ssem):
    c = lax.axis_index("c")
    s = lax.axis_index("s")
    wid = c * NS + s
    _per_subcore_slice(s, NH, lambda off, n: pltpu.sync_copy(
        z128_h.at[pl.ds(off, n)], acc_sp.at[pl.ds(off, n)]))
    pltpu.sync_copy(zd2_h, den_t)
    pltpu.sync_copy(bnd_h, bnd_v)
    plsc.subcore_barrier()
    bvec = bnd_v[...]
    zv = jnp.full((16,), 0, jnp.int32)
    ov = jnp.full((16,), 1, jnp.int32)
    base = c * NH

    def blk(k, carry):
        bg = s + NS * k
        row = bg * NB
        pltpu.sync_copy(src_h.at[pl.ds(row, NB)], src_v)
        pltpu.sync_copy(dst_h.at[pl.ds(row, NB)], dst_v)
        def gath(i):
            p = i % 2
            return [pltpu.async_copy(att_h.at[src_v.at[i]], asrc[p], gsem[p]),
                    pltpu.async_copy(att_h.at[dst_v.at[i]], adst[p], gsem[p]),
                    pltpu.async_copy(h2_h.at[src_v.at[i]], hbuf[p], gsem[p])]

        def work(i):
            p = i % 2
            for j in range(5):
                eidx = lax.iota(jnp.int32, 16) + j * 16
                dstv = dst_v[i, pl.ds(j * 16, 16)]
                lidx = dstv - base
                inh = (lidx >= 0) & (lidx < NH)
                sv = plsc.load_gather(asrc[p], [eidx, zv])
                dv = plsc.load_gather(adst[p], [eidx, ov])
                t = sv + dv
                t = jnp.maximum(t, 0.2 * t)
                w = jnp.exp(t - bvec) * inh.astype(jnp.float32)
                w_v[p, pl.ds(j * 16, 16)] = w
                idx_v[i, pl.ds(j * 16, 16)] = jnp.where(inh, lidx, 0)
                plsc.addupdate_scatter(den_t, [zv, dstv], w)
            pv = jnp.full((16,), p, jnp.int32)

            def rowfn(r, cr):
                for u in range(4):
                    wb = plsc.load_gather(
                        w_v, [pv, jnp.broadcast_to(4 * r + u, (16,))])
                    for kk in range(8):
                        hbuf[p][4 * r + u, pl.ds(kk * 16, 16)] = (
                            hbuf[p][4 * r + u, pl.ds(kk * 16, 16)] * wb)
                return cr

            lax.fori_loop(0, CH // 4, rowfn, 0)
            return pltpu.async_copy(hbuf[p], acc_sp.at[idx_v.at[i]],
                                    ssem[p], add=True)

        sd = [None] * NB
        gd = gath(0)
        for i in range(1, NB + 1):
            if i >= 2:
                sd[i - 2].wait()
            nd = gath(i) if i < NB else []
            for d in gd:
                d.wait()
            sd[i - 1] = work(i - 1)
            gd = nd
        sd[NB - 1].wait()
        return carry

    cnt = jnp.where(s < NBLK - 31 * NS, 32, 31)
    lax.fori_loop(0, cnt, blk, 0)
    pltpu.sync_copy(den_t, den_h.at[wid])
    plsc.subcore_barrier()
    _per_subcore_slice(s, NH, lambda off, n: pltpu.sync_copy(
        acc_sp.at[pl.ds(off, n)], acc_h.at[pl.ds(base + off, n)]))


_scc = functools.partial(
    pl.kernel,
    _scc_body,
    out_type=(_f32((N, 128)), _f32((NW, 1, N))),
    mesh=_mesh,
    compiler_params=_params,
    scratch_types=[
        pltpu.VMEM((NB, CH), jnp.int32),
        pltpu.VMEM((NB, CH), jnp.int32),
        pltpu.VMEM((NB, CH), jnp.int32),
        pltpu.VMEM((2, CH), jnp.float32),
        (pltpu.VMEM((CH, 128), jnp.float32), pltpu.VMEM((CH, 128), jnp.float32)),
        (pltpu.VMEM((CH, 128), jnp.float32), pltpu.VMEM((CH, 128), jnp.float32)),
        (pltpu.VMEM((CH, 128), jnp.float32), pltpu.VMEM((CH, 128), jnp.float32)),
        pltpu.VMEM((16,), jnp.float32),
        pltpu.VMEM((1, N), jnp.float32),
        pltpu.VMEM_SHARED((NH, 128), jnp.float32),
        (pltpu.SemaphoreType.DMA, pltpu.SemaphoreType.DMA),
        (pltpu.SemaphoreType.DMA, pltpu.SemaphoreType.DMA),
    ],
)()


def _tc3_body(acc_ref, den_ref, att2_ref, h2_ref, b2_ref, bnd_ref, out_ref):
    t = att2_ref[:, 0:1] + att2_ref[:, 1:2]
    t = jnp.maximum(t, 0.2 * t)
    sw = jnp.exp(t - bnd_ref[0, 0])
    den = jnp.sum(den_ref[...], axis=1, keepdims=True) + sw + 1e-16
    out_ref[...] = ((acc_ref[...] + sw * h2_ref[...]) / den + b2_ref[...])


def _tc3(acc2, den2p, att2, h2, b2, bnd2):
    return pl.pallas_call(
        _tc3_body,
        grid=(25,),
        in_specs=[
            pl.BlockSpec((400, 128), lambda i: (i, 0)),
            pl.BlockSpec((400, NW), lambda i: (i, 0)),
            pl.BlockSpec((400, 128), lambda i: (i, 0)),
            pl.BlockSpec((400, 128), lambda i: (i, 0)),
            pl.BlockSpec((1, 128), lambda i: (0, 0)),
            pl.BlockSpec((1, 1), lambda i: (0, 0)),
        ],
        out_specs=pl.BlockSpec((400, 128), lambda i: (i, 0)),
        out_shape=_f32((N, 128)),
    )(acc2, den2p, att2, h2, b2, bnd2)


def kernel(x, edge_index, W1, att_src1, att_dst1, b1,
           W2, att_src2, att_dst2, b2):
    src = edge_index[0].astype(jnp.int32).reshape(E // CH, CH)
    dst = edge_index[1].astype(jnp.int32).reshape(E // CH, CH)

    rows1 = jnp.arange(1024)
    A1 = jnp.zeros((1024, 128), jnp.float32)
    A1 = A1.at[rows1, rows1 // 128].set(att_src1.reshape(-1))
    A1 = A1.at[rows1, 8 + rows1 // 128].set(att_dst1.reshape(-1))
    rows2 = jnp.arange(128)
    A2 = jnp.zeros((128, 128), jnp.float32)
    A2 = A2.at[rows2, 0].set(att_src2.reshape(-1))
    A2 = A2.at[rows2, 1].set(att_dst2.reshape(-1))

    zden = jnp.zeros((1, 8 * N), jnp.float32)
    zd2 = jnp.zeros((1, N), jnp.float32)
    z128 = jnp.zeros((N, 128), jnp.float32)

    h1, att1, bnd1 = _tc1(x, W1, A1)
    bv1 = jnp.max(bnd1, axis=0)
    bsh8 = bv1[:8] + bv1[8:16]
    bsh1 = jnp.concatenate([bsh8, jnp.zeros((8,), jnp.float32)])
    wt, den1p = _sca(src, dst, att1,
                     jnp.broadcast_to(bsh8[:, None], (8, 16)), zden)
    den1p = den1p.reshape(NW, N, 8)
    out1 = _scb(src, dst, wt.reshape(8, E // CH, CH),
                h1.reshape(N * 8, 128), z128)
    h2, att2, bnd2 = _tc2(out1, den1p, att1, h1, W2, A2,
                          b1.reshape(1, 1024), bsh1.reshape(1, 16))
    bv2 = jnp.max(bnd2, axis=0)
    bound2 = bv2[0] + bv2[1]
    acc2, den2p = _scc(src, dst, att2, h2, jnp.full((16,), bound2),
                       z128, zd2)
    den2t = den2p.reshape(NW, N).T
    return _tc3(acc2, den2t, att2, h2, b2.reshape(1, 128),
                bound2.reshape(1, 1))


# speedup vs baseline: 1.1477x; 1.0487x over previous
import functools

import jax
import jax.numpy as jnp
from jax import lax
from jax.experimental import pallas as pl
from jax.experimental.pallas import tpu as pltpu
from jax.experimental.pallas import tpu_sc as plsc

N = 10000
NH = N // 2
E = 320000
NC = 2
NS = 16
NW = NC * NS
CH = 80
NB = 8
B = NB * CH
NBLK = E // B

_mesh = plsc.VectorSubcoreMesh(
    core_axis_name="c", subcore_axis_name="s", num_cores=NC, num_subcores=NS)
_params = pltpu.CompilerParams(needs_layout_passes=False)


def _f32(shape):
    return jax.ShapeDtypeStruct(shape, jnp.float32)


def _per_subcore_slice(s, total, copyfn):
    big = (total // NS + 7) // 8 * 8
    last = total - (NS - 1) * big

    @pl.when(s < NS - 1)
    def _():
        copyfn(s * big, big)

    @pl.when(s == NS - 1)
    def _():
        copyfn((NS - 1) * big, last)


def _tc1_body(x_ref, w_ref, a_ref, h_ref, att_ref, bnd_ref):
    i = pl.program_id(0)
    h = jnp.dot(x_ref[...], w_ref[...], preferred_element_type=jnp.float32)
    att = jnp.dot(h, a_ref[...], preferred_element_type=jnp.float32)
    h_ref[...] = h
    att_ref[...] = att
    bm = jnp.broadcast_to(jnp.max(att, axis=0, keepdims=True), (8, 128))

    @pl.when(i == 0)
    def _():
        bnd_ref[...] = bm

    @pl.when(i != 0)
    def _():
        bnd_ref[...] = jnp.maximum(bnd_ref[...], bm)


def _tc1(x, W1, A1):
    return pl.pallas_call(
        _tc1_body,
        grid=(25,),
        in_specs=[
            pl.BlockSpec((400, 128), lambda i: (i, 0)),
            pl.BlockSpec((128, 1024), lambda i: (0, 0)),
            pl.BlockSpec((1024, 128), lambda i: (0, 0)),
        ],
        out_specs=[
            pl.BlockSpec((400, 1024), lambda i: (i, 0)),
            pl.BlockSpec((400, 128), lambda i: (i, 0)),
            pl.BlockSpec((8, 128), lambda i: (0, 0)),
        ],
        out_shape=[_f32((N, 1024)), _f32((N, 128)), _f32((8, 128))],
    )(x, W1, A1)


def _sca_body(src_h, dst_h, att_h, bnd_h, zden_h,
              wt_h, den_h,
              src_v, dst_v, asrc, adst, wtv, bnd_v, den_t, gsem):
    c = lax.axis_index("c")
    s = lax.axis_index("s")
    wid = c * NS + s
    pltpu.sync_copy(zden_h, den_t)
    pltpu.sync_copy(bnd_h, bnd_v)
    bvecs = [bnd_v[h, pl.ds(0, 16)] for h in range(8)]

    def blk(k, carry):
        bg = wid + NW * k
        row = bg * NB
        pltpu.sync_copy(src_h.at[pl.ds(row, NB)], src_v)
        pltpu.sync_copy(dst_h.at[pl.ds(row, NB)], dst_v)
        def gath(i):
            p = i % 2
            return [pltpu.async_copy(att_h.at[src_v.at[i]], asrc[p], gsem[p]),
                    pltpu.async_copy(att_h.at[dst_v.at[i]], adst[p], gsem[p])]

        def work(i):
            p = i % 2
            for j in range(5):
                eidx = lax.iota(jnp.int32, 16) + j * 16
                dstv = dst_v[i, pl.ds(j * 16, 16)]
                zrow = jnp.full((16,), 0, jnp.int32)
                for h in range(8):
                    hv = jnp.full((16,), h, jnp.int32)
                    sv = plsc.load_gather(asrc[p], [eidx, hv])
                    dv = plsc.load_gather(adst[p],
                                          [eidx, jnp.full((16,), 8 + h,
                                                          jnp.int32)])
                    t = sv + dv
                    t = jnp.maximum(t, 0.2 * t)
                    w = jnp.exp(t - bvecs[h])
                    wtv[h, pl.ds(i * CH + j * 16, 16)] = w
                    plsc.addupdate_scatter(den_t, [zrow, dstv * 8 + h], w)

        gd = gath(0)
        for i in range(1, NB + 1):
            nd = gath(i) if i < NB else []
            for d in gd:
                d.wait()
            work(i - 1)
            gd = nd
        pltpu.sync_copy(wtv, wt_h.at[:, pl.ds(bg * B, B)])
        return carry

    cnt = jnp.where(wid < NBLK - 15 * NW, 16, 15)
    lax.fori_loop(0, cnt, blk, 0)
    pltpu.sync_copy(den_t, den_h.at[wid])


_sca = functools.partial(
    pl.kernel,
    _sca_body,
    out_type=(_f32((8, E)), _f32((NW, 1, 8 * N))),
    mesh=_mesh,
    compiler_params=_params,
    scratch_types=[
        pltpu.VMEM((NB, CH), jnp.int32),
        pltpu.VMEM((NB, CH), jnp.int32),
        (pltpu.VMEM((CH, 128), jnp.float32), pltpu.VMEM((CH, 128), jnp.float32)),
        (pltpu.VMEM((CH, 128), jnp.float32), pltpu.VMEM((CH, 128), jnp.float32)),
        pltpu.VMEM((8, B), jnp.float32),
        pltpu.VMEM((8, 16), jnp.float32),
        pltpu.VMEM((1, 8 * N), jnp.float32),
        (pltpu.SemaphoreType.DMA, pltpu.SemaphoreType.DMA),
    ],
)()


def _scb_body(src_h, dst_h, wt_h, h1_h, z128_h,
              out_h,
              src_v, dst_v, idx_v, w_v, hbuf, hbuf2, acc_sp, gsem, ssem):
    c = lax.axis_index("c")
    s = lax.axis_index("s")
    for hl in range(4):
        hg = c * 4 + hl
        _per_subcore_slice(s, N, lambda off, n: pltpu.sync_copy(
            z128_h.at[pl.ds(off, n)], acc_sp.at[pl.ds(off, n)]))
        plsc.subcore_barrier()

        def blk(k, carry):
            bg = s + NS * k
            row = bg * NB
            pltpu.sync_copy(src_h.at[pl.ds(row, NB)], src_v)
            pltpu.sync_copy(dst_h.at[pl.ds(row, NB)], dst_v)
            pltpu.sync_copy(wt_h.at[hg, pl.ds(row, NB)], w_v)
            for j in range(B // 16):
                r, q = j // 5, (j % 5) * 16
                sv = src_v[r, pl.ds(q, 16)]
                idx_v[r, pl.ds(q, 16)] = sv * 8 + hg

            def scale(buf, i):
                iv = jnp.full((16,), i, jnp.int32)

                def rowfn(r, cr):
                    for u in range(4):
                        wb = plsc.load_gather(
                            w_v, [iv, jnp.broadcast_to(4 * r + u, (16,))])
                        for kk in range(8):
                            buf[4 * r + u, pl.ds(kk * 16, 16)] = (
                                buf[4 * r + u, pl.ds(kk * 16, 16)] * wb)
                    return cr

                lax.fori_loop(0, CH // 4, rowfn, 0)

            bufs = (hbuf, hbuf2)
            gd = [None] * NB
            sd = [None] * NB
            gd[0] = pltpu.async_copy(h1_h.at[idx_v.at[0]], bufs[0], gsem[0])
            for i in range(1, NB + 1):
                if i >= 2:
                    sd[i - 2].wait()
                if i < NB:
                    gd[i] = pltpu.async_copy(
                        h1_h.at[idx_v.at[i]], bufs[i % 2], gsem[i % 2])
                gd[i - 1].wait()
                scale(bufs[(i - 1) % 2], i - 1)
                sd[i - 1] = pltpu.async_copy(
                    bufs[(i - 1) % 2], acc_sp.at[dst_v.at[i - 1]],
                    ssem[(i - 1) % 2], add=True)
            sd[NB - 1].wait()
            return carry

        cnt = jnp.where(s < NBLK - 31 * NS, 32, 31)
        lax.fori_loop(0, cnt, blk, 0)
        plsc.subcore_barrier()
        _per_subcore_slice(s, N, lambda off, n: pltpu.sync_copy(
            acc_sp.at[pl.ds(off, n)], out_h.at[hg, pl.ds(off, n)]))
        plsc.subcore_barrier()


_scb = functools.partial(
    pl.kernel,
    _scb_body,
    out_type=_f32((8, N, 128)),
    mesh=_mesh,
    compiler_params=_params,
    scratch_types=[
        pltpu.VMEM((NB, CH), jnp.int32),
        pltpu.VMEM((NB, CH), jnp.int32),
        pltpu.VMEM((NB, CH), jnp.int32),
        pltpu.VMEM((NB, CH), jnp.float32),
        pltpu.VMEM((CH, 128), jnp.float32),
        pltpu.VMEM((CH, 128), jnp.float32),
        pltpu.VMEM_SHARED((N, 128), jnp.float32),
        (pltpu.SemaphoreType.DMA, pltpu.SemaphoreType.DMA),
        (pltpu.SemaphoreType.DMA, pltpu.SemaphoreType.DMA),
    ],
)()


def _tc2_body(o1_ref, den_ref, att_ref, h1_ref, w2_ref, a2_ref, b1_ref,
              bsh_ref, h2_ref, att2_ref, bnd_ref):
    i = pl.program_id(0)
    att = att_ref[...]
    pieces = []
    for h in range(8):
        t = att[:, h:h + 1] + att[:, 8 + h:9 + h]
        t = jnp.maximum(t, 0.2 * t)
        sw = jnp.exp(t - bsh_ref[0, h])
        den = jnp.sum(den_ref[:, :, h:h + 1], axis=0) + sw + 1e-16
        num = o1_ref[h] + sw * h1_ref[:, h * 128:(h + 1) * 128]
        o = num / den + b1_ref[:, h * 128:(h + 1) * 128]
        pieces.append(jnp.where(o > 0, o, jnp.exp(jnp.minimum(o, 0.0)) - 1.0))
    x2 = jnp.concatenate(pieces, axis=1)
    h2 = jnp.dot(x2, w2_ref[...], preferred_element_type=jnp.float32)
    att2 = jnp.dot(h2, a2_ref[...], preferred_element_type=jnp.float32)
    h2_ref[...] = h2
    att2_ref[...] = att2
    bm = jnp.broadcast_to(jnp.max(att2, axis=0, keepdims=True), (8, 128))

    @pl.when(i == 0)
    def _():
        bnd_ref[...] = bm

    @pl.when(i != 0)
    def _():
        bnd_ref[...] = jnp.maximum(bnd_ref[...], bm)


def _tc2(out1, den1p, att1, h1, W2, A2, b1, bsh1):
    return pl.pallas_call(
        _tc2_body,
        grid=(25,),
        in_specs=[
            pl.BlockSpec((8, 400, 128), lambda i: (0, i, 0)),
            pl.BlockSpec((NW, 400, 8), lambda i: (0, i, 0)),
            pl.BlockSpec((400, 128), lambda i: (i, 0)),
            pl.BlockSpec((400, 1024), lambda i: (i, 0)),
            pl.BlockSpec((1024, 128), lambda i: (0, 0)),
            pl.BlockSpec((128, 128), lambda i: (0, 0)),
            pl.BlockSpec((1, 1024), lambda i: (0, 0)),
            pl.BlockSpec((1, 16), lambda i: (0, 0)),
        ],
        out_specs=[
            pl.BlockSpec((400, 128), lambda i: (i, 0)),
            pl.BlockSpec((400, 128), lambda i: (i, 0)),
            pl.BlockSpec((8, 128), lambda i: (0, 0)),
        ],
        out_shape=[_f32((N, 128)), _f32((N, 128)), _f32((8, 128))],
    )(out1, den1p, att1, h1, W2, A2, b1, bsh1)


def _scc_body(src_h, dst_h, att_h, h2_h, bnd_h, z128_h, zd2_h,
              acc_h, den_h,
              src_v, dst_v, idx_v, w_v, asrc, adst, hbuf, bnd_v, den_t,
              acc_sp, gsem, ssem):
    c = lax.axis_index("c")
    s = lax.axis_index("s")
    wid = c * NS + s
    _per_subcore_slice(s, NH, lambda off, n: pltpu.sync_copy(
        z128_h.at[pl.ds(off, n)], acc_sp.at[pl.ds(off, n)]))
    pltpu.sync_copy(zd2_h, den_t)
    pltpu.sync_copy(bnd_h, bnd_v)
    plsc.subcore_barrier()
    bvec = bnd_v[...]
    zv = jnp.full((16,), 0, jnp.int32)
    ov = jnp.full((16,), 1, jnp.int32)
    base = c * NH

    def blk(k, carry):
        bg = s + NS * k
        row = bg * NB
        pltpu.sync_copy(src_h.at[pl.ds(row, NB)], src_v)
        pltpu.sync_copy(dst_h.at[pl.ds(row, NB)], dst_v)
        def gath(i):
            p = i % 2
            return [pltpu.async_copy(att_h.at[src_v.at[i]], asrc[p], gsem[p]),
                    pltpu.async_copy(att_h.at[dst_v.at[i]], adst[p], gsem[p]),
                    pltpu.async_copy(h2_h.at[src_v.at[i]], hbuf[p], gsem[p])]

        def work(i):
            p = i % 2
            for j in range(5):
                eidx = lax.iota(jnp.int32, 16) + j * 16
                dstv = dst_v[i, pl.ds(j * 16, 16)]
                lidx = dstv - base
                inh = (lidx >= 0) & (lidx < NH)
                sv = plsc.load_gather(asrc[p], [eidx, zv])
                dv = plsc.load_gather(adst[p], [eidx, ov])
                t = sv + dv
                t = jnp.maximum(t, 0.2 * t)
                w = jnp.exp(t - bvec) * inh.astype(jnp.float32)
                w_v[p, pl.ds(j * 16, 16)] = w
                idx_v[i, pl.ds(j * 16, 16)] = jnp.where(inh, lidx, 0)
                plsc.addupdate_scatter(den_t, [zv, dstv], w)
            pv = jnp.full((16,), p, jnp.int32)

            def rowfn(r, cr):
                for u in range(4):
                    wb = plsc.load_gather(
                        w_v, [pv, jnp.broadcast_to(4 * r + u, (16,))])
                    for kk in range(8):
                        hbuf[p][4 * r + u, pl.ds(kk * 16, 16)] = (
                            hbuf[p][4 * r + u, pl.ds(kk * 16, 16)] * wb)
                return cr

            lax.fori_loop(0, CH // 4, rowfn, 0)
            return pltpu.async_copy(hbuf[p], acc_sp.at[idx_v.at[i]],
                                    ssem[p], add=True)

        sd = [None] * NB
        gd = gath(0)
        for i in range(1, NB + 1):
            if i >= 2:
                sd[i - 2].wait()
            nd = gath(i) if i < NB else []
            for d in gd:
                d.wait()
            sd[i - 1] = work(i - 1)
            gd = nd
        sd[NB - 1].wait()
        return carry

    cnt = jnp.where(s < NBLK - 31 * NS, 32, 31)
    lax.fori_loop(0, cnt, blk, 0)
    pltpu.sync_copy(den_t, den_h.at[wid])
    plsc.subcore_barrier()
    _per_subcore_slice(s, NH, lambda off, n: pltpu.sync_copy(
        acc_sp.at[pl.ds(off, n)], acc_h.at[pl.ds(base + off, n)]))


_scc = functools.partial(
    pl.kernel,
    _scc_body,
    out_type=(_f32((N, 128)), _f32((NW, 1, N))),
    mesh=_mesh,
    compiler_params=_params,
    scratch_types=[
        pltpu.VMEM((NB, CH), jnp.int32),
        pltpu.VMEM((NB, CH), jnp.int32),
        pltpu.VMEM((NB, CH), jnp.int32),
        pltpu.VMEM((2, CH), jnp.float32),
        (pltpu.VMEM((CH, 128), jnp.float32), pltpu.VMEM((CH, 128), jnp.float32)),
        (pltpu.VMEM((CH, 128), jnp.float32), pltpu.VMEM((CH, 128), jnp.float32)),
        (pltpu.VMEM((CH, 128), jnp.float32), pltpu.VMEM((CH, 128), jnp.float32)),
        pltpu.VMEM((16,), jnp.float32),
        pltpu.VMEM((1, N), jnp.float32),
        pltpu.VMEM_SHARED((NH, 128), jnp.float32),
        (pltpu.SemaphoreType.DMA, pltpu.SemaphoreType.DMA),
        (pltpu.SemaphoreType.DMA, pltpu.SemaphoreType.DMA),
    ],
)()


def _tc3_body(acc_ref, den_ref, att2_ref, h2_ref, b2_ref, bnd_ref, out_ref):
    t = att2_ref[:, 0:1] + att2_ref[:, 1:2]
    t = jnp.maximum(t, 0.2 * t)
    sw = jnp.exp(t - bnd_ref[0, 0])
    den = jnp.sum(den_ref[...], axis=1, keepdims=True) + sw + 1e-16
    out_ref[...] = ((acc_ref[...] + sw * h2_ref[...]) / den + b2_ref[...])


def _tc3(acc2, den2p, att2, h2, b2, bnd2):
    return pl.pallas_call(
        _tc3_body,
        grid=(25,),
        in_specs=[
            pl.BlockSpec((400, 128), lambda i: (i, 0)),
            pl.BlockSpec((400, NW), lambda i: (i, 0)),
            pl.BlockSpec((400, 128), lambda i: (i, 0)),
            pl.BlockSpec((400, 128), lambda i: (i, 0)),
            pl.BlockSpec((1, 128), lambda i: (0, 0)),
            pl.BlockSpec((1, 1), lambda i: (0, 0)),
        ],
        out_specs=pl.BlockSpec((400, 128), lambda i: (i, 0)),
        out_shape=_f32((N, 128)),
    )(acc2, den2p, att2, h2, b2, bnd2)


def kernel(x, edge_index, W1, att_src1, att_dst1, b1,
           W2, att_src2, att_dst2, b2):
    src = edge_index[0].astype(jnp.int32).reshape(E // CH, CH)
    dst = edge_index[1].astype(jnp.int32).reshape(E // CH, CH)

    rows1 = jnp.arange(1024)
    A1 = jnp.zeros((1024, 128), jnp.float32)
    A1 = A1.at[rows1, rows1 // 128].set(att_src1.reshape(-1))
    A1 = A1.at[rows1, 8 + rows1 // 128].set(att_dst1.reshape(-1))
    rows2 = jnp.arange(128)
    A2 = jnp.zeros((128, 128), jnp.float32)
    A2 = A2.at[rows2, 0].set(att_src2.reshape(-1))
    A2 = A2.at[rows2, 1].set(att_dst2.reshape(-1))

    zden = jnp.zeros((1, 8 * N), jnp.float32)
    zd2 = jnp.zeros((1, N), jnp.float32)
    z128 = jnp.zeros((N, 128), jnp.float32)

    h1, att1, bnd1 = _tc1(x, W1, A1)
    bv1 = jnp.max(bnd1, axis=0)
    bsh8 = bv1[:8] + bv1[8:16]
    bsh1 = jnp.concatenate([bsh8, jnp.zeros((8,), jnp.float32)])
    wt, den1p = _sca(src, dst, att1,
                     jnp.broadcast_to(bsh8[:, None], (8, 16)), zden)
    den1p = den1p.reshape(NW, N, 8)
    out1 = _scb(src, dst, wt.reshape(8, E // CH, CH),
                h1.reshape(N * 8, 128), z128)
    h2, att2, bnd2 = _tc2(out1, den1p, att1, h1, W2, A2,
                          b1.reshape(1, 1024), bsh1.reshape(1, 16))
    bv2 = jnp.max(bnd2, axis=0)
    bound2 = bv2[0] + bv2[1]
    acc2, den2p = _scc(src, dst, att2, h2, jnp.full((16,), bound2),
                       z128, zd2)
    den2t = den2p.reshape(NW, N).T
    return _tc3(acc2, den2t, att2, h2, b2.reshape(1, 128),
                bound2.reshape(1, 1))
